# baseline (speedup 1.0000x reference)
import functools

import jax
import jax.numpy as jnp
from jax import lax
from jax.experimental import pallas as pl
from jax.experimental.pallas import tpu as pltpu
from jax.experimental.pallas import tpu_sc as plsc

LOCAL_EXPERTS = 8
TOPK = 2
ROW_TILE = 256
INTER_BLK = 2048
GATE_TILE = 512

_NC, _NS = 2, 16
NWORKERS = _NC * _NS
GATHER_CHUNK = 16


def _gate_body(x_ref, w_ref, b_ref, o_ref):
    o_ref[...] = (
        jnp.dot(x_ref[...], w_ref[...], preferred_element_type=jnp.float32)
        + b_ref[...]
    )


def _gate_logits(x, gate_w, gate_b):
    t, h = x.shape
    tot = gate_w.shape[1]
    grid = (t // GATE_TILE,)
    return pl.pallas_call(
        _gate_body,
        grid=grid,
        in_specs=[
            pl.BlockSpec((GATE_TILE, h), lambda i: (i, 0)),
            pl.BlockSpec((h, tot), lambda i: (0, 0)),
            pl.BlockSpec((1, tot), lambda i: (0, 0)),
        ],
        out_specs=pl.BlockSpec((GATE_TILE, tot), lambda i: (i, 0)),
        out_shape=jax.ShapeDtypeStruct((t, tot), jnp.float32),
    )(x, gate_w, gate_b.reshape(1, tot))


def _sc_gather(table, idx):
    n = idx.shape[0]
    h = table.shape[1]
    per_w = n // NWORKERS
    n_chunks = per_w // GATHER_CHUNK
    mesh = plsc.VectorSubcoreMesh(core_axis_name="c", subcore_axis_name="s")

    @functools.partial(
        pl.kernel,
        out_type=jax.ShapeDtypeStruct((n, h), jnp.float32),
        mesh=mesh,
        scratch_types=[
            pltpu.VMEM((per_w,), jnp.int32),
            pltpu.VMEM((GATHER_CHUNK, h), jnp.float32),
            pltpu.SemaphoreType.DMA,
        ],
    )
    def gather_kernel(table_hbm, idx_hbm, out_hbm, idx_v, rows_v, sem):
        wid = lax.axis_index("s") * _NC + lax.axis_index("c")
        base = wid * per_w
        pltpu.sync_copy(idx_hbm.at[pl.ds(base, per_w)], idx_v)

        def body(c, carry):
            off = c * GATHER_CHUNK
            ivec = idx_v[pl.ds(off, GATHER_CHUNK)]
            pltpu.async_copy(table_hbm.at[ivec], rows_v, sem).wait()
            pltpu.sync_copy(rows_v, out_hbm.at[pl.ds(base + off, GATHER_CHUNK)])
            return carry

        lax.fori_loop(0, n_chunks, body, 0)

    return gather_kernel(table, idx)


def _sc_combine(ys, pos_a, pos_b):
    t = pos_a.shape[0]
    h = ys.shape[1]
    per_w = t // NWORKERS
    n_chunks = per_w // GATHER_CHUNK
    vecs_per_chunk = GATHER_CHUNK * h // 16
    mesh = plsc.VectorSubcoreMesh(core_axis_name="c", subcore_axis_name="s")

    @functools.partial(
        pl.kernel,
        out_type=jax.ShapeDtypeStruct((t, h), jnp.float32),
        mesh=mesh,
        scratch_types=[
            pltpu.VMEM((per_w,), jnp.int32),
            pltpu.VMEM((per_w,), jnp.int32),
            pltpu.VMEM((GATHER_CHUNK, h), jnp.float32),
            pltpu.VMEM((GATHER_CHUNK, h), jnp.float32),
            pltpu.SemaphoreType.DMA,
            pltpu.SemaphoreType.DMA,
        ],
    )
    def combine_kernel(ys_hbm, pa_hbm, pb_hbm, out_hbm,
                       pa_v, pb_v, buf_a, buf_b, sem_a, sem_b):
        wid = lax.axis_index("s") * _NC + lax.axis_index("c")
        base = wid * per_w
        pltpu.sync_copy(pa_hbm.at[pl.ds(base, per_w)], pa_v)
        pltpu.sync_copy(pb_hbm.at[pl.ds(base, per_w)], pb_v)
        vecs_per_row = h // 16

        def body(c, carry):
            off = c * GATHER_CHUNK
            ia = pa_v[pl.ds(off, GATHER_CHUNK)]
            ib = pb_v[pl.ds(off, GATHER_CHUNK)]
            cp_a = pltpu.async_copy(ys_hbm.at[ia], buf_a, sem_a)
            cp_b = pltpu.async_copy(ys_hbm.at[ib], buf_b, sem_b)
            cp_a.wait()
            cp_b.wait()

            def add_body(f, carry2):
                row = f // vecs_per_row
                s = (f % vecs_per_row) * 16
                buf_a[row, pl.ds(s, 16)] = (
                    buf_a[row, pl.ds(s, 16)] + buf_b[row, pl.ds(s, 16)]
                )
                return carry2

            lax.fori_loop(0, vecs_per_chunk, add_body, 0)
            pltpu.sync_copy(buf_a, out_hbm.at[pl.ds(base + off, GATHER_CHUNK)])
            return carry

        lax.fori_loop(0, n_chunks, body, 0)

    return combine_kernel(ys, pos_a, pos_b)


def _cast_body(i_ref, o_ref):
    o_ref[...] = i_ref[...].astype(jnp.bfloat16)


def _cast_bf16(w, rows_blk):
    flat = w.reshape(-1, w.shape[-1])
    n, c = flat.shape
    out = pl.pallas_call(
        _cast_body,
        grid=(n // rows_blk,),
        in_specs=[pl.BlockSpec((rows_blk, c), lambda i: (i, 0))],
        out_specs=pl.BlockSpec((rows_blk, c), lambda i: (i, 0)),
        out_shape=jax.ShapeDtypeStruct((n, c), jnp.bfloat16),
    )(flat)
    return out.reshape(w.shape)


def _fc1_body(te_ref, xs_ref, w1_ref, b1_ref, hs_ref, w1bf_ref):
    i = pl.program_id(0)
    r = pl.program_id(1)
    prev = te_ref[jnp.maximum(r - 1, 0)]
    fresh = jnp.logical_or(r == 0, te_ref[r] != prev)

    @pl.when(fresh)
    def _recast():
        w1bf_ref[...] = w1_ref[0].astype(jnp.bfloat16)

    x_bf = xs_ref[...].astype(jnp.bfloat16)
    h = (
        jnp.dot(x_bf, w1bf_ref[...], preferred_element_type=jnp.float32)
        + b1_ref[0]
    )
    hs_ref[...] = jax.nn.gelu(h, approximate=True).astype(jnp.bfloat16)


def _fc2_body(te_ref, hs_ref, w2_ref, b2_ref, wr_ref, o_ref, acc_ref,
              *, n_inter):
    i = pl.program_id(1)

    @pl.when(i == 0)
    def _init():
        acc_ref[...] = jnp.zeros_like(acc_ref)

    acc_ref[...] += jnp.dot(hs_ref[...], w2_ref[0],
                            preferred_element_type=jnp.float32)

    @pl.when(i == n_inter - 1)
    def _fin():
        o_ref[...] = (acc_ref[...] + b2_ref[0]) * wr_ref[...]


def _grouped_mlp(xs, tile_expert, fc1_w, fc1_b, fc2_w, fc2_b, row_w):
    r_pad, h = xs.shape
    e, _, inter = fc1_w.shape
    n_r = r_pad // ROW_TILE
    n_i = inter // INTER_BLK

    fc1_spec = pltpu.PrefetchScalarGridSpec(
        num_scalar_prefetch=1,
        grid=(n_i, n_r),
        in_specs=[
            pl.BlockSpec((ROW_TILE, h), lambda i, r, te: (r, 0)),
            pl.BlockSpec((1, h, INTER_BLK), lambda i, r, te: (te[r], 0, i)),
            pl.BlockSpec((1, 1, INTER_BLK), lambda i, r, te: (te[r], 0, i)),
        ],
        out_specs=pl.BlockSpec((ROW_TILE, INTER_BLK), lambda i, r, te: (r, i)),
        scratch_shapes=[pltpu.VMEM((h, INTER_BLK), jnp.bfloat16)],
    )
    hs = pl.pallas_call(
        _fc1_body,
        grid_spec=fc1_spec,
        out_shape=jax.ShapeDtypeStruct((r_pad, inter), jnp.bfloat16),
        compiler_params=pltpu.CompilerParams(
            dimension_semantics=("arbitrary", "arbitrary"),
        ),
    )(tile_expert, xs, fc1_w, fc1_b.reshape(e, 1, inter))

    fc2_spec = pltpu.PrefetchScalarGridSpec(
        num_scalar_prefetch=1,
        grid=(n_r, n_i),
        in_specs=[
            pl.BlockSpec((ROW_TILE, INTER_BLK), lambda r, i, te: (r, i)),
            pl.BlockSpec((1, INTER_BLK, h), lambda r, i, te: (te[r], i, 0)),
            pl.BlockSpec((1, 1, h), lambda r, i, te: (te[r], 0, 0)),
            pl.BlockSpec((ROW_TILE, 1), lambda r, i, te: (r, 0)),
        ],
        out_specs=pl.BlockSpec((ROW_TILE, h), lambda r, i, te: (r, 0)),
        scratch_shapes=[pltpu.VMEM((ROW_TILE, h), jnp.float32)],
    )
    return pl.pallas_call(
        functools.partial(_fc2_body, n_inter=n_i),
        grid_spec=fc2_spec,
        out_shape=jax.ShapeDtypeStruct((r_pad, h), jnp.float32),
        compiler_params=pltpu.CompilerParams(
            dimension_semantics=("arbitrary", "arbitrary"),
        ),
    )(tile_expert, hs, _cast_bf16(fc2_w, 2048), fc2_b.reshape(e, 1, h),
      row_w.reshape(r_pad, 1))


def _route(logits, alpha, r_pad):
    t, _ = logits.shape
    r = t * TOPK
    top2_val, top2_idx = lax.top_k(logits, TOPK)
    local = jnp.mod(top2_idx, LOCAL_EXPERTS)
    mx = jnp.max(top2_val, axis=-1, keepdims=True)
    ez = jnp.exp(top2_val - mx)
    gate = ez / jnp.sum(ez, axis=-1, keepdims=True)
    wgt = gate * alpha[local]

    ef = local.reshape(-1).astype(jnp.int32)
    tf = (jnp.arange(r, dtype=jnp.int32) // TOPK)
    wf = wgt.reshape(-1)
    order = jnp.argsort(ef, stable=True)
    e_s, t_s, w_s = ef[order], tf[order], wf[order]

    counts = jnp.bincount(ef, length=LOCAL_EXPERTS)
    padded = ((counts + ROW_TILE - 1) // ROW_TILE) * ROW_TILE
    pad_end = jnp.cumsum(padded)
    pad_start = pad_end - padded
    grp_start = jnp.cumsum(counts) - counts
    dest = (jnp.arange(r, dtype=jnp.int32)
            - grp_start[e_s].astype(jnp.int32)
            + pad_start[e_s].astype(jnp.int32))

    row_token = jnp.zeros((r_pad,), jnp.int32).at[dest].set(t_s)
    row_w = jnp.zeros((r_pad,), jnp.float32).at[dest].set(w_s)
    pos = jnp.zeros((r,), jnp.int32).at[order].set(dest).reshape(t, TOPK)

    n_tiles = r_pad // ROW_TILE
    tile_starts = jnp.arange(n_tiles, dtype=jnp.int32) * ROW_TILE
    tile_expert = jnp.clip(
        jnp.searchsorted(pad_end, tile_starts, side="right"),
        0, LOCAL_EXPERTS - 1).astype(jnp.int32)

    return row_token, row_w, pos[:, 0], pos[:, 1], tile_expert


def kernel(hidden_states, gate_w, gate_b, alpha, fc1_w, fc1_b, fc2_w, fc2_b):
    b, s, h = hidden_states.shape
    t = b * s
    r_pad = t * TOPK + LOCAL_EXPERTS * ROW_TILE
    x = hidden_states.reshape(t, h)

    logits = _gate_logits(x, gate_w, gate_b)
    row_token, row_w, pos_a, pos_b, tile_expert = _route(logits, alpha, r_pad)
    xs = _sc_gather(x, row_token)
    ys = _grouped_mlp(xs, tile_expert, fc1_w, fc1_b, fc2_w, fc2_b, row_w)
    out = _sc_combine(ys, pos_a, pos_b)
    return out.reshape(b, s, h)


# speedup vs baseline: 1.0677x; 1.0677x over previous
import functools

import jax
import jax.numpy as jnp
from jax import lax
from jax.experimental import pallas as pl
from jax.experimental.pallas import tpu as pltpu
from jax.experimental.pallas import tpu_sc as plsc

LOCAL_EXPERTS = 8
TOPK = 2
ROW_TILE = 256
INTER_BLK = 2048
GATE_TILE = 512

_NC, _NS = 2, 16
NWORKERS = _NC * _NS
GATHER_CHUNK = 16


def _gate_body(x_ref, w_ref, b_ref, o_ref):
    o_ref[...] = (
        jnp.dot(x_ref[...], w_ref[...], preferred_element_type=jnp.float32)
        + b_ref[...]
    )


def _gate_logits(x, gate_w, gate_b):
    t, h = x.shape
    tot = gate_w.shape[1]
    grid = (t // GATE_TILE,)
    return pl.pallas_call(
        _gate_body,
        grid=grid,
        in_specs=[
            pl.BlockSpec((GATE_TILE, h), lambda i: (i, 0)),
            pl.BlockSpec((h, tot), lambda i: (0, 0)),
            pl.BlockSpec((1, tot), lambda i: (0, 0)),
        ],
        out_specs=pl.BlockSpec((GATE_TILE, tot), lambda i: (i, 0)),
        out_shape=jax.ShapeDtypeStruct((t, tot), jnp.float32),
    )(x, gate_w, gate_b.reshape(1, tot))


def _sc_gather(table, idx):
    n = idx.shape[0]
    h = table.shape[1]
    per_w = n // NWORKERS
    n_chunks = per_w // GATHER_CHUNK
    mesh = plsc.VectorSubcoreMesh(core_axis_name="c", subcore_axis_name="s")

    @functools.partial(
        pl.kernel,
        out_type=jax.ShapeDtypeStruct((n, h), jnp.float32),
        mesh=mesh,
        scratch_types=[
            pltpu.VMEM((per_w,), jnp.int32),
            pltpu.VMEM((GATHER_CHUNK, h), jnp.float32),
            pltpu.SemaphoreType.DMA,
        ],
    )
    def gather_kernel(table_hbm, idx_hbm, out_hbm, idx_v, rows_v, sem):
        wid = lax.axis_index("s") * _NC + lax.axis_index("c")
        base = wid * per_w
        pltpu.sync_copy(idx_hbm.at[pl.ds(base, per_w)], idx_v)

        def body(c, carry):
            off = c * GATHER_CHUNK
            ivec = idx_v[pl.ds(off, GATHER_CHUNK)]
            pltpu.async_copy(table_hbm.at[ivec], rows_v, sem).wait()
            pltpu.sync_copy(rows_v, out_hbm.at[pl.ds(base + off, GATHER_CHUNK)])
            return carry

        lax.fori_loop(0, n_chunks, body, 0)

    return gather_kernel(table, idx)


def _sc_combine(ys, pos_a, pos_b):
    t = pos_a.shape[0]
    h = ys.shape[1]
    per_w = t // NWORKERS
    n_chunks = per_w // GATHER_CHUNK
    vecs_per_chunk = GATHER_CHUNK * h // 16
    mesh = plsc.VectorSubcoreMesh(core_axis_name="c", subcore_axis_name="s")

    @functools.partial(
        pl.kernel,
        out_type=jax.ShapeDtypeStruct((t, h), jnp.float32),
        mesh=mesh,
        scratch_types=[
            pltpu.VMEM((per_w,), jnp.int32),
            pltpu.VMEM((per_w,), jnp.int32),
            pltpu.VMEM((GATHER_CHUNK, h), jnp.float32),
            pltpu.VMEM((GATHER_CHUNK, h), jnp.float32),
            pltpu.SemaphoreType.DMA,
            pltpu.SemaphoreType.DMA,
        ],
    )
    def combine_kernel(ys_hbm, pa_hbm, pb_hbm, out_hbm,
                       pa_v, pb_v, buf_a, buf_b, sem_a, sem_b):
        wid = lax.axis_index("s") * _NC + lax.axis_index("c")
        base = wid * per_w
        pltpu.sync_copy(pa_hbm.at[pl.ds(base, per_w)], pa_v)
        pltpu.sync_copy(pb_hbm.at[pl.ds(base, per_w)], pb_v)
        vecs_per_row = h // 16

        def body(c, carry):
            off = c * GATHER_CHUNK
            ia = pa_v[pl.ds(off, GATHER_CHUNK)]
            ib = pb_v[pl.ds(off, GATHER_CHUNK)]
            cp_a = pltpu.async_copy(ys_hbm.at[ia], buf_a, sem_a)
            cp_b = pltpu.async_copy(ys_hbm.at[ib], buf_b, sem_b)
            cp_a.wait()
            cp_b.wait()

            @plsc.parallel_loop(0, vecs_per_chunk, unroll=8)
            def add_body(f):
                row = f // vecs_per_row
                s = (f % vecs_per_row) * 16
                buf_a[row, pl.ds(s, 16)] = (
                    buf_a[row, pl.ds(s, 16)] + buf_b[row, pl.ds(s, 16)]
                )
            pltpu.sync_copy(buf_a, out_hbm.at[pl.ds(base + off, GATHER_CHUNK)])
            return carry

        lax.fori_loop(0, n_chunks, body, 0)

    return combine_kernel(ys, pos_a, pos_b)


def _cast_body(i_ref, o_ref):
    o_ref[...] = i_ref[...].astype(jnp.bfloat16)


def _cast_bf16(w, rows_blk):
    flat = w.reshape(-1, w.shape[-1])
    n, c = flat.shape
    out = pl.pallas_call(
        _cast_body,
        grid=(n // rows_blk,),
        in_specs=[pl.BlockSpec((rows_blk, c), lambda i: (i, 0))],
        out_specs=pl.BlockSpec((rows_blk, c), lambda i: (i, 0)),
        out_shape=jax.ShapeDtypeStruct((n, c), jnp.bfloat16),
    )(flat)
    return out.reshape(w.shape)


def _mlp_body(te_ref, xs_ref, w1_ref, b1_ref, w2_ref, b2_ref, wr_ref,
              o_ref, acc_ref, xbf_ref, *, n_inter):
    i = pl.program_id(1)

    @pl.when(i == 0)
    def _init():
        acc_ref[...] = jnp.zeros_like(acc_ref)
        xbf_ref[...] = xs_ref[...].astype(jnp.bfloat16)

    h = (
        jnp.dot(xbf_ref[...], w1_ref[0], preferred_element_type=jnp.float32)
        + b1_ref[0]
    )
    h = jax.nn.gelu(h, approximate=True)
    acc_ref[...] += jnp.dot(h.astype(jnp.bfloat16), w2_ref[0],
                            preferred_element_type=jnp.float32)

    @pl.when(i == n_inter - 1)
    def _fin():
        o_ref[...] = (acc_ref[...] + b2_ref[0]) * wr_ref[...]


def _grouped_mlp(xs, tile_expert, fc1_w, fc1_b, fc2_w, fc2_b, row_w):
    r_pad, h = xs.shape
    e, _, inter = fc1_w.shape
    n_r = r_pad // ROW_TILE
    n_i = inter // INTER_BLK
    grid_spec = pltpu.PrefetchScalarGridSpec(
        num_scalar_prefetch=1,
        grid=(n_r, n_i),
        in_specs=[
            pl.BlockSpec((ROW_TILE, h), lambda r, i, te: (r, 0)),
            pl.BlockSpec((1, h, INTER_BLK), lambda r, i, te: (te[r], 0, i)),
            pl.BlockSpec((1, 1, INTER_BLK), lambda r, i, te: (te[r], 0, i)),
            pl.BlockSpec((1, INTER_BLK, h), lambda r, i, te: (te[r], i, 0)),
            pl.BlockSpec((1, 1, h), lambda r, i, te: (te[r], 0, 0)),
            pl.BlockSpec((ROW_TILE, 1), lambda r, i, te: (r, 0)),
        ],
        out_specs=pl.BlockSpec((ROW_TILE, h), lambda r, i, te: (r, 0)),
        scratch_shapes=[pltpu.VMEM((ROW_TILE, h), jnp.float32),
                        pltpu.VMEM((ROW_TILE, h), jnp.bfloat16)],
    )
    return pl.pallas_call(
        functools.partial(_mlp_body, n_inter=n_i),
        grid_spec=grid_spec,
        out_shape=jax.ShapeDtypeStruct((r_pad, h), jnp.float32),
        compiler_params=pltpu.CompilerParams(
            dimension_semantics=("arbitrary", "arbitrary"),
        ),
    )(tile_expert, xs,
      _cast_bf16(fc1_w, 512), fc1_b.reshape(e, 1, inter),
      _cast_bf16(fc2_w, 2048), fc2_b.reshape(e, 1, h),
      row_w.reshape(r_pad, 1))


def _route(logits, alpha, r_pad):
    t, _ = logits.shape
    r = t * TOPK
    top2_val, top2_idx = lax.top_k(logits, TOPK)
    local = jnp.mod(top2_idx, LOCAL_EXPERTS)
    mx = jnp.max(top2_val, axis=-1, keepdims=True)
    ez = jnp.exp(top2_val - mx)
    gate = ez / jnp.sum(ez, axis=-1, keepdims=True)
    wgt = gate * alpha[local]

    ef = local.reshape(-1).astype(jnp.int32)
    tf = (jnp.arange(r, dtype=jnp.int32) // TOPK)
    wf = wgt.reshape(-1)
    order = jnp.argsort(ef, stable=True)
    e_s, t_s, w_s = ef[order], tf[order], wf[order]

    counts = jnp.bincount(ef, length=LOCAL_EXPERTS).astype(jnp.int32)
    padded = ((counts + ROW_TILE - 1) // ROW_TILE) * ROW_TILE
    pad_end = jnp.cumsum(padded)
    pad_start = pad_end - padded
    grp_start = jnp.cumsum(counts) - counts

    d_idx = jnp.arange(r_pad, dtype=jnp.int32)
    e_of_d = jnp.clip(
        jnp.sum(d_idx[:, None] >= pad_end[None, :], axis=1),
        0, LOCAL_EXPERTS - 1)
    slot = d_idx - pad_start[e_of_d]
    valid = slot < counts[e_of_d]
    r_of_d = jnp.clip(slot + grp_start[e_of_d], 0, r - 1)
    row_token = jnp.where(valid, t_s[r_of_d], 0).astype(jnp.int32)
    row_w = jnp.where(valid, w_s[r_of_d], 0.0)

    dest = (jnp.arange(r, dtype=jnp.int32) - grp_start[e_s] + pad_start[e_s])
    inv = jnp.argsort(order)
    pos = dest[inv].reshape(t, TOPK)

    n_tiles = r_pad // ROW_TILE
    tile_starts = jnp.arange(n_tiles, dtype=jnp.int32) * ROW_TILE
    tile_expert = jnp.clip(
        jnp.sum(tile_starts[:, None] >= pad_end[None, :], axis=1),
        0, LOCAL_EXPERTS - 1).astype(jnp.int32)

    return row_token, row_w, pos[:, 0], pos[:, 1], tile_expert


def kernel(hidden_states, gate_w, gate_b, alpha, fc1_w, fc1_b, fc2_w, fc2_b):
    b, s, h = hidden_states.shape
    t = b * s
    r_pad = t * TOPK + LOCAL_EXPERTS * ROW_TILE
    x = hidden_states.reshape(t, h)

    logits = _gate_logits(x, gate_w, gate_b)
    row_token, row_w, pos_a, pos_b, tile_expert = _route(logits, alpha, r_pad)
    xs = _sc_gather(x, row_token)
    ys = _grouped_mlp(xs, tile_expert, fc1_w, fc1_b, fc2_w, fc2_b, row_w)
    out = _sc_combine(ys, pos_a, pos_b)
    return out.reshape(b, s, h)
